# baseline (speedup 1.0000x reference)
import functools

import jax
import jax.numpy as jnp
from jax import lax
from jax.experimental import pallas as pl
from jax.experimental.pallas import tpu as pltpu
from jax.experimental.pallas import tpu_sc as plsc

_K = 8
_OHEM_RATIO = 3.0
_NEG_FILL = -1e30
_BISECT_ITERS = 40

_LVL = (
    dict(base=0, nsteps=40, nc=5, bh=32, s=160, total=8 * 160 * 160,
         sl1=False),
    dict(base=40, nsteps=8, nc=1, bh=80, s=80, total=8 * 80 * 80, sl1=True),
    dict(base=48, nsteps=8, nc=1, bh=40, s=40, total=8 * 40 * 40, sl1=True),
)
_NSTEPS = 56


def _softplus(x):
    return jnp.maximum(x, 0.0) + jnp.log1p(jnp.exp(-jnp.abs(x)))


def _level_body(t, lvl, refs):
    base, nsteps, total = lvl["base"], lvl["nsteps"], lvl["total"]
    sl1 = lvl["sl1"]
    if sl1:
        (cls_ref, msk_ref, rx_ref, ry_ref, gx_ref, gy_ref,
         out_ref, accm_ref, accx_ref, accy_ref, negce_ref, sel_ref) = refs
    else:
        (cls_ref, msk_ref, out_ref, accm_ref, negce_ref, sel_ref) = refs
    step = t - base

    @pl.when(step == 0)
    def _init():
        accm_ref[...] = jnp.zeros_like(accm_ref)
        if sl1:
            accx_ref[...] = jnp.zeros_like(accx_ref)
            accy_ref[...] = jnp.zeros_like(accy_ref)

    c = cls_ref[0]
    m = msk_ref[0].astype(jnp.float32)
    tr = m[0]
    tcl = m[1]
    tm = m[2]
    ce_tr = _softplus((c[0] - c[1]) * (2.0 * tr - 1.0))
    ce_tcl = _softplus((c[2] - c[3]) * (2.0 * tcl - 1.0))

    pos = tr * tm
    neg = tm - pos
    negce = jnp.where(neg > 0.0, ce_tr, _NEG_FILL)
    negce_ref[step] = negce

    accm_ref[0] += tm + 64.0 * pos
    accm_ref[1] += ce_tr * pos
    accm_ref[2] += ce_tcl * pos
    accm_ref[3] += ce_tcl
    accm_ref[4] += ce_tr * neg

    if sl1:
        wm = pos * (tr + tcl) * 0.125
        dx = jnp.abs(gx_ref[0] - rx_ref[0])
        mx = jnp.minimum(dx, 1.0)
        accx_ref[...] += wm * jnp.sum(dx - mx + 0.5 * mx * mx, axis=0)
        dy = jnp.abs(gy_ref[0] - ry_ref[0])
        my = jnp.minimum(dy, 1.0)
        accy_ref[...] += wm * jnp.sum(dy - my + 0.5 * my * my, axis=0)

    @pl.when(step == nsteps - 1)
    def _finalize():
        packed = accm_ref[0]
        pos_pl = jnp.floor(packed * (1.0 / 64.0))
        n_pos = jnp.sum(pos_pl)
        n_neg_all = jnp.sum(packed - 64.0 * pos_pl) - n_pos
        s_ce_pos = jnp.sum(accm_ref[1])
        s_tcl_pos = jnp.sum(accm_ref[2])
        s_tcl_all = jnp.sum(accm_ref[3])
        s_neg_all = jnp.sum(accm_ref[4])

        has_pos = n_pos > 0.0
        n_neg = jnp.where(has_pos,
                          jnp.minimum(n_neg_all,
                                      jnp.floor(_OHEM_RATIO * n_pos)),
                          100.0)
        eff = jnp.minimum(n_neg, n_neg_all)
        need_select = eff < n_neg_all

        @pl.when(need_select)
        def _bisect():
            v = negce_ref[...]
            maxv = jnp.maximum(jnp.max(v), 0.0)

            def body(_, carry):
                lo, hi = carry
                mid = 0.5 * (lo + hi)
                cnt = jnp.sum((v > mid).astype(jnp.float32))
                take_lo = cnt >= eff
                return (jnp.where(take_lo, mid, lo),
                        jnp.where(take_lo, hi, mid))

            lo, hi = lax.fori_loop(0, _BISECT_ITERS, body, (0.0, maxv))
            cnt_hi = jnp.sum((v > hi).astype(jnp.float32))
            sum_hi = jnp.sum(jnp.where(v > hi, v, 0.0))
            sel_ref[0] = sum_hi + (eff - cnt_hi) * hi

        loss_neg = jnp.where(need_select, sel_ref[0], s_neg_all)
        loss_pos = jnp.where(has_pos, s_ce_pos, 0.0)
        l_tr = (loss_pos + loss_neg) / (n_pos + n_neg)

        tcl_pos = s_tcl_pos / jnp.maximum(n_pos, 1.0)
        tcl_neg = (s_tcl_all - s_tcl_pos) / jnp.maximum(float(total) - n_pos,
                                                        1.0)
        l_tcl = jnp.where(has_pos, tcl_pos + 0.5 * tcl_neg, 0.0)

        if sl1:
            sx = jnp.sum(accx_ref[...])
            sy = jnp.sum(accy_ref[...])
            denom = jnp.maximum(n_pos * float(_K), 1.0)
            out_ref[0] += l_tr
            out_ref[1] += l_tcl
            out_ref[2] += jnp.where(has_pos, sx / denom, 0.0)
            out_ref[3] += jnp.where(has_pos, sy / denom, 0.0)
        else:
            out_ref[0] = l_tr
            out_ref[1] = l_tcl
            out_ref[2] = 0.0
            out_ref[3] = 0.0
            out_ref[4] = n_pos


def _fused_kernel(c3, m3,
                  c4, m4, rx4, ry4, gx4, gy4,
                  c5, m5, rx5, ry5, gx5, gy5,
                  out_ref,
                  accm3, negce3,
                  accm4, accx4, accy4, negce4,
                  accm5, accx5, accy5, negce5,
                  sel_ref):
    t = pl.program_id(0)
    ops = ((c3, m3, out_ref, accm3, negce3, sel_ref),
           (c4, m4, rx4, ry4, gx4, gy4, out_ref,
            accm4, accx4, accy4, negce4, sel_ref),
           (c5, m5, rx5, ry5, gx5, gy5, out_ref,
            accm5, accx5, accy5, negce5, sel_ref))
    for lvl, refs in zip(_LVL, ops):
        lo, hi = lvl["base"], lvl["base"] + lvl["nsteps"]

        @pl.when((t >= lo) & (t < hi))
        def _run(lvl=lvl, refs=refs):
            _level_body(t, lvl, refs)


def _level_specs(lvl):
    base, nc, bh, s = lvl["base"], lvl["nc"], lvl["bh"], lvl["s"]
    last_i = lvl["nsteps"] // nc - 1
    last_j = nc - 1

    def im(cblk):
        def f(t):
            sidx = t - base
            i = jnp.clip(sidx // nc, 0, last_i)
            j = jnp.clip(sidx % nc, 0, last_j)
            j = jnp.where((t >= base) & (sidx // nc <= last_i), j, last_j)
            j = jnp.where(t < base, 0, j)
            i = jnp.where(t < base, 0, i)
            return (i, cblk, j, 0)
        return f

    specs = [
        pl.BlockSpec((1, 4, bh, s), im(0)),
        pl.BlockSpec((1, 3, bh, s), im(0)),
    ]
    if lvl["sl1"]:
        specs += [
            pl.BlockSpec((1, _K, bh, s), im(0)),
            pl.BlockSpec((1, _K, bh, s), im(1)),
            pl.BlockSpec((1, _K, bh, s), im(0)),
            pl.BlockSpec((1, _K, bh, s), im(1)),
        ]
    return specs


def _level_scratch(lvl):
    bh, s, nsteps = lvl["bh"], lvl["s"], lvl["nsteps"]
    scratch = [pltpu.VMEM((5, bh, s), jnp.float32)]
    if lvl["sl1"]:
        scratch += [pltpu.VMEM((bh, s), jnp.float32),
                    pltpu.VMEM((bh, s), jnp.float32)]
    scratch += [pltpu.VMEM((nsteps, bh, s), jnp.float32)]
    return scratch


def _tc_call(p3_cls, p3_mask, p4_cls, p4_mask, p4_reg, p4_map,
             p5_cls, p5_mask, p5_reg, p5_map):
    specs = (_level_specs(_LVL[0]) + _level_specs(_LVL[1])
             + _level_specs(_LVL[2]))
    scratch = (_level_scratch(_LVL[0]) + _level_scratch(_LVL[1])
               + _level_scratch(_LVL[2]) + [pltpu.SMEM((1,), jnp.float32)])
    return pl.pallas_call(
        _fused_kernel,
        grid=(_NSTEPS,),
        in_specs=specs,
        out_specs=pl.BlockSpec(memory_space=pltpu.SMEM),
        out_shape=jax.ShapeDtypeStruct((5,), jnp.float32),
        scratch_shapes=scratch,
    )(p3_cls, p3_mask,
      p4_cls, p4_mask, p4_reg, p4_reg, p4_map, p4_map,
      p5_cls, p5_mask, p5_reg, p5_reg, p5_map, p5_map)


_SC_NC = 2
_SC_NS = 16
_SC_UNITS_PER_TILE = 5
_SC_ROWS = 8
_SC_S = 160
_SC_VECS = _SC_ROWS * _SC_S // 16


def _sc_sl1_kernel(reg_hbm, map_hbm, msk_hbm, out_hbm,
                   trb, tclb, tmb, chb, wmb, accv, sem):
    cid = lax.axis_index("c")
    sid = lax.axis_index("s")
    wid = sid * _SC_NC + cid

    accv[0, :] = jnp.zeros((16,), jnp.float32)
    accv[1, :] = jnp.zeros((16,), jnp.float32)

    for u in range(_SC_UNITS_PER_TILE):
        unit = wid * _SC_UNITS_PER_TILE + u
        n = unit // 20
        g = unit % 20
        r0 = g * _SC_ROWS
        hs = [
            pltpu.async_copy(msk_hbm.at[n, 0, pl.ds(r0, _SC_ROWS), :],
                             trb, sem),
            pltpu.async_copy(msk_hbm.at[n, 1, pl.ds(r0, _SC_ROWS), :],
                             tclb, sem),
            pltpu.async_copy(msk_hbm.at[n, 2, pl.ds(r0, _SC_ROWS), :],
                             tmb, sem),
        ]
        for j in range(2 * _K):
            hs.append(pltpu.async_copy(
                reg_hbm.at[n, j, pl.ds(r0, _SC_ROWS), :], chb.at[2 * j],
                sem))
            hs.append(pltpu.async_copy(
                map_hbm.at[n, j, pl.ds(r0, _SC_ROWS), :], chb.at[2 * j + 1],
                sem))
        for h in hs:
            h.wait()

        def _wm_body(r, _):
            i = r // 10
            o = (r % 10) * 16
            trv = trb[i, pl.ds(o, 16)].astype(jnp.float32)
            tclv = tclb[i, pl.ds(o, 16)].astype(jnp.float32)
            tmv = tmb[i, pl.ds(o, 16)].astype(jnp.float32)
            wmb[i, pl.ds(o, 16)] = trv * tmv * (trv + tclv) * 0.125
            return 0

        lax.fori_loop(0, _SC_VECS, _wm_body, 0)

        def _ch_body(r, _):
            i = r // 10
            o = (r % 10) * 16
            w = wmb[i, pl.ds(o, 16)]
            sxa = jnp.zeros((16,), jnp.float32)
            sya = jnp.zeros((16,), jnp.float32)
            for j in range(2 * _K):
                a = chb[2 * j, i, pl.ds(o, 16)]
                b = chb[2 * j + 1, i, pl.ds(o, 16)]
                d = jnp.abs(b - a)
                mn = jnp.minimum(d, 1.0)
                sl = d - mn + 0.5 * mn * mn
                if j < _K:
                    sxa = sxa + sl
                else:
                    sya = sya + sl
            accv[0, :] += w * sxa
            accv[1, :] += w * sya
            return 0

        lax.fori_loop(0, _SC_VECS, _ch_body, 0)

    pltpu.sync_copy(accv, out_hbm.at[wid])


_SC_CACHE = []


def _sc_sl1(reg, mp, msk):
    if not _SC_CACHE:
        _SC_CACHE.append(functools.partial(
            pl.kernel,
            out_type=jax.ShapeDtypeStruct((_SC_NC * _SC_NS, 2, 16),
                                          jnp.float32),
            mesh=plsc.VectorSubcoreMesh(core_axis_name="c",
                                        subcore_axis_name="s",
                                        num_cores=_SC_NC,
                                        num_subcores=_SC_NS),
            scratch_types=[
                pltpu.VMEM((_SC_ROWS, _SC_S), jnp.int32),
                pltpu.VMEM((_SC_ROWS, _SC_S), jnp.int32),
                pltpu.VMEM((_SC_ROWS, _SC_S), jnp.int32),
                pltpu.VMEM((4 * _K, _SC_ROWS, _SC_S), jnp.float32),
                pltpu.VMEM((_SC_ROWS, _SC_S), jnp.float32),
                pltpu.VMEM((2, 16), jnp.float32),
                pltpu.SemaphoreType.DMA,
            ],
        )(_sc_sl1_kernel))
    return _SC_CACHE[0](reg, mp, msk)


def kernel(p3_cls, p3_reg, p3_mask, p3_map,
           p4_cls, p4_reg, p4_mask, p4_map,
           p5_cls, p5_reg, p5_mask, p5_map):
    sc = _sc_sl1(p3_reg, p3_map, p3_mask)
    tc = _tc_call(p3_cls, p3_mask, p4_cls, p4_mask, p4_reg, p4_map,
                  p5_cls, p5_mask, p5_reg, p5_map)
    sx3 = jnp.sum(sc[:, 0, :])
    sy3 = jnp.sum(sc[:, 1, :])
    np3 = tc[4]
    has3 = np3 > 0.0
    den3 = jnp.maximum(np3 * float(_K), 1.0)
    l_rx = tc[2] + jnp.where(has3, sx3 / den3, 0.0)
    l_ry = tc[3] + jnp.where(has3, sy3 / den3, 0.0)
    return jnp.stack([tc[0], tc[1], l_rx, l_ry])


# speedup vs baseline: 1.0315x; 1.0315x over previous
import functools

import jax
import jax.numpy as jnp
from jax import lax
from jax.experimental import pallas as pl
from jax.experimental.pallas import tpu as pltpu
from jax.experimental.pallas import tpu_sc as plsc

_K = 8
_OHEM_RATIO = 3.0
_NEG_FILL = -1e30
_BISECT_ITERS = 40

_LVL = (
    dict(base=0, nsteps=40, nc=5, bh=32, s=160, total=8 * 160 * 160,
         sl1=False),
    dict(base=40, nsteps=8, nc=1, bh=80, s=80, total=8 * 80 * 80, sl1=True),
    dict(base=48, nsteps=8, nc=1, bh=40, s=40, total=8 * 40 * 40, sl1=True),
)
_NSTEPS = 56


def _softplus(x):
    return jnp.maximum(x, 0.0) + jnp.log1p(jnp.exp(-jnp.abs(x)))


def _level_body(t, lvl, refs):
    base, nsteps, total = lvl["base"], lvl["nsteps"], lvl["total"]
    sl1 = lvl["sl1"]
    if sl1:
        (cls_ref, msk_ref, rx_ref, ry_ref, gx_ref, gy_ref,
         out_ref, accm_ref, accx_ref, accy_ref, negce_ref, sel_ref) = refs
    else:
        (cls_ref, msk_ref, out_ref, accm_ref, negce_ref, sel_ref) = refs
    step = t - base

    @pl.when(step == 0)
    def _init():
        accm_ref[...] = jnp.zeros_like(accm_ref)
        if sl1:
            accx_ref[...] = jnp.zeros_like(accx_ref)
            accy_ref[...] = jnp.zeros_like(accy_ref)

    c = cls_ref[0]
    m = msk_ref[0].astype(jnp.float32)
    tr = m[0]
    tcl = m[1]
    tm = m[2]
    ce_tr = _softplus((c[0] - c[1]) * (2.0 * tr - 1.0))
    ce_tcl = _softplus((c[2] - c[3]) * (2.0 * tcl - 1.0))

    pos = tr * tm
    neg = tm - pos
    negce = jnp.where(neg > 0.0, ce_tr, _NEG_FILL)
    negce_ref[step] = negce

    accm_ref[0] += tm + 64.0 * pos
    accm_ref[1] += ce_tr * pos
    accm_ref[2] += ce_tcl * pos
    accm_ref[3] += ce_tcl
    accm_ref[4] += ce_tr * neg

    if sl1:
        wm = pos * (tr + tcl) * 0.125
        dx = jnp.abs(gx_ref[0] - rx_ref[0])
        mx = jnp.minimum(dx, 1.0)
        accx_ref[...] += wm * jnp.sum(dx - mx + 0.5 * mx * mx, axis=0)
        dy = jnp.abs(gy_ref[0] - ry_ref[0])
        my = jnp.minimum(dy, 1.0)
        accy_ref[...] += wm * jnp.sum(dy - my + 0.5 * my * my, axis=0)

    @pl.when(step == nsteps - 1)
    def _finalize():
        packed = accm_ref[0]
        pos_pl = jnp.floor(packed * (1.0 / 64.0))
        n_pos = jnp.sum(pos_pl)
        n_neg_all = jnp.sum(packed - 64.0 * pos_pl) - n_pos
        s_ce_pos = jnp.sum(accm_ref[1])
        s_tcl_pos = jnp.sum(accm_ref[2])
        s_tcl_all = jnp.sum(accm_ref[3])
        s_neg_all = jnp.sum(accm_ref[4])

        has_pos = n_pos > 0.0
        n_neg = jnp.where(has_pos,
                          jnp.minimum(n_neg_all,
                                      jnp.floor(_OHEM_RATIO * n_pos)),
                          100.0)
        eff = jnp.minimum(n_neg, n_neg_all)
        need_select = eff < n_neg_all

        @pl.when(need_select)
        def _bisect():
            v = negce_ref[...]
            maxv = jnp.maximum(jnp.max(v), 0.0)

            def body(_, carry):
                lo, hi = carry
                mid = 0.5 * (lo + hi)
                cnt = jnp.sum((v > mid).astype(jnp.float32))
                take_lo = cnt >= eff
                return (jnp.where(take_lo, mid, lo),
                        jnp.where(take_lo, hi, mid))

            lo, hi = lax.fori_loop(0, _BISECT_ITERS, body, (0.0, maxv))
            cnt_hi = jnp.sum((v > hi).astype(jnp.float32))
            sum_hi = jnp.sum(jnp.where(v > hi, v, 0.0))
            sel_ref[0] = sum_hi + (eff - cnt_hi) * hi

        loss_neg = jnp.where(need_select, sel_ref[0], s_neg_all)
        loss_pos = jnp.where(has_pos, s_ce_pos, 0.0)
        l_tr = (loss_pos + loss_neg) / (n_pos + n_neg)

        tcl_pos = s_tcl_pos / jnp.maximum(n_pos, 1.0)
        tcl_neg = (s_tcl_all - s_tcl_pos) / jnp.maximum(float(total) - n_pos,
                                                        1.0)
        l_tcl = jnp.where(has_pos, tcl_pos + 0.5 * tcl_neg, 0.0)

        if sl1:
            sx = jnp.sum(accx_ref[...])
            sy = jnp.sum(accy_ref[...])
            denom = jnp.maximum(n_pos * float(_K), 1.0)
            out_ref[0] += l_tr
            out_ref[1] += l_tcl
            out_ref[2] += jnp.where(has_pos, sx / denom, 0.0)
            out_ref[3] += jnp.where(has_pos, sy / denom, 0.0)
        else:
            out_ref[0] = l_tr
            out_ref[1] = l_tcl
            out_ref[2] = 0.0
            out_ref[3] = 0.0
            out_ref[4] = n_pos


def _fused_kernel(c3, m3,
                  c4, m4, rx4, ry4, gx4, gy4,
                  c5, m5, rx5, ry5, gx5, gy5,
                  out_ref,
                  accm3, negce3,
                  accm4, accx4, accy4, negce4,
                  accm5, accx5, accy5, negce5,
                  sel_ref):
    t = pl.program_id(0)
    ops = ((c3, m3, out_ref, accm3, negce3, sel_ref),
           (c4, m4, rx4, ry4, gx4, gy4, out_ref,
            accm4, accx4, accy4, negce4, sel_ref),
           (c5, m5, rx5, ry5, gx5, gy5, out_ref,
            accm5, accx5, accy5, negce5, sel_ref))
    for lvl, refs in zip(_LVL, ops):
        lo, hi = lvl["base"], lvl["base"] + lvl["nsteps"]

        @pl.when((t >= lo) & (t < hi))
        def _run(lvl=lvl, refs=refs):
            _level_body(t, lvl, refs)


def _level_specs(lvl):
    base, nc, bh, s = lvl["base"], lvl["nc"], lvl["bh"], lvl["s"]
    last_i = lvl["nsteps"] // nc - 1
    last_j = nc - 1

    def im(cblk):
        def f(t):
            sidx = t - base
            i = jnp.clip(sidx // nc, 0, last_i)
            j = jnp.clip(sidx % nc, 0, last_j)
            j = jnp.where((t >= base) & (sidx // nc <= last_i), j, last_j)
            j = jnp.where(t < base, 0, j)
            i = jnp.where(t < base, 0, i)
            return (i, cblk, j, 0)
        return f

    specs = [
        pl.BlockSpec((1, 4, bh, s), im(0)),
        pl.BlockSpec((1, 3, bh, s), im(0)),
    ]
    if lvl["sl1"]:
        specs += [
            pl.BlockSpec((1, _K, bh, s), im(0)),
            pl.BlockSpec((1, _K, bh, s), im(1)),
            pl.BlockSpec((1, _K, bh, s), im(0)),
            pl.BlockSpec((1, _K, bh, s), im(1)),
        ]
    return specs


def _level_scratch(lvl):
    bh, s, nsteps = lvl["bh"], lvl["s"], lvl["nsteps"]
    scratch = [pltpu.VMEM((5, bh, s), jnp.float32)]
    if lvl["sl1"]:
        scratch += [pltpu.VMEM((bh, s), jnp.float32),
                    pltpu.VMEM((bh, s), jnp.float32)]
    scratch += [pltpu.VMEM((nsteps, bh, s), jnp.float32)]
    return scratch


def _tc_call(p3_cls, p3_mask, p4_cls, p4_mask, p4_reg, p4_map,
             p5_cls, p5_mask, p5_reg, p5_map):
    specs = (_level_specs(_LVL[0]) + _level_specs(_LVL[1])
             + _level_specs(_LVL[2]))
    scratch = (_level_scratch(_LVL[0]) + _level_scratch(_LVL[1])
               + _level_scratch(_LVL[2]) + [pltpu.SMEM((1,), jnp.float32)])
    return pl.pallas_call(
        _fused_kernel,
        grid=(_NSTEPS,),
        in_specs=specs,
        out_specs=pl.BlockSpec(memory_space=pltpu.SMEM),
        out_shape=jax.ShapeDtypeStruct((5,), jnp.float32),
        scratch_shapes=scratch,
    )(p3_cls, p3_mask,
      p4_cls, p4_mask, p4_reg, p4_reg, p4_map, p4_map,
      p5_cls, p5_mask, p5_reg, p5_reg, p5_map, p5_map)


_SC_NC = 2
_SC_NS = 16
_SC_UNITS_PER_TILE = 5
_SC_ROWS = 8
_SC_S = 160
_SC_VECS = _SC_ROWS * _SC_S // 16


def _sc_sl1_kernel(reg_hbm, map_hbm, msk_hbm, out_hbm,
                   mskb, wmb, gbuf, accv, sem_m, sem0, sem1):
    cid = lax.axis_index("c")
    sid = lax.axis_index("s")
    wid = sid * _SC_NC + cid
    gsems = (sem0, sem1)

    def unit_loc(u):
        unit = wid * _SC_UNITS_PER_TILE + u
        return unit // 20, (unit % 20) * _SC_ROWS

    def issue_masks(u):
        n, r0 = unit_loc(u)
        return [pltpu.async_copy(
            msk_hbm.at[n, ch, pl.ds(r0, _SC_ROWS), :],
            mskb.at[ch], sem_m) for ch in range(3)]

    def issue_group(u, gi, buf):
        n, r0 = unit_loc(u)
        hs = []
        for jj in range(4):
            j = gi * 4 + jj
            hs.append(pltpu.async_copy(
                reg_hbm.at[n, j, pl.ds(r0, _SC_ROWS), :],
                gbuf.at[buf, 2 * jj], gsems[buf]))
            hs.append(pltpu.async_copy(
                map_hbm.at[n, j, pl.ds(r0, _SC_ROWS), :],
                gbuf.at[buf, 2 * jj + 1], gsems[buf]))
        return hs

    sx = jnp.zeros((16,), jnp.float32)
    sy = jnp.zeros((16,), jnp.float32)
    hs_m = issue_masks(0)
    for u in range(_SC_UNITS_PER_TILE):
        hs_g = issue_group(u, 0, 0)
        for h in hs_m:
            h.wait()

        def _wm_body(r, _):
            i = r // 10
            o = (r % 10) * 16
            trv = mskb[0, i, pl.ds(o, 16)].astype(jnp.float32)
            tclv = mskb[1, i, pl.ds(o, 16)].astype(jnp.float32)
            tmv = mskb[2, i, pl.ds(o, 16)].astype(jnp.float32)
            wmb[i, pl.ds(o, 16)] = trv * tmv * (trv + tclv) * 0.125
            return 0

        lax.fori_loop(0, _SC_VECS, _wm_body, 0)

        for gi in range(4):
            buf = gi % 2
            if gi < 3:
                hs_next = issue_group(u, gi + 1, 1 - buf)
            elif u + 1 < _SC_UNITS_PER_TILE:
                hs_next = []
                hs_m = issue_masks(u + 1)
            else:
                hs_next = []
            for h in hs_g:
                h.wait()
            hs_g = hs_next
            is_x = gi < 2

            def _g_body(r, carry, buf=buf):
                acc = carry
                i = r // 10
                o = (r % 10) * 16
                w = wmb[i, pl.ds(o, 16)]
                s = jnp.zeros((16,), jnp.float32)
                for jj in range(4):
                    a = gbuf[buf, 2 * jj, i, pl.ds(o, 16)]
                    b = gbuf[buf, 2 * jj + 1, i, pl.ds(o, 16)]
                    d = jnp.abs(b - a)
                    mn = jnp.minimum(d, 1.0)
                    s = s + (d - mn + 0.5 * mn * mn)
                return acc + w * s

            if is_x:
                sx = lax.fori_loop(0, _SC_VECS, _g_body, sx)
            else:
                sy = lax.fori_loop(0, _SC_VECS, _g_body, sy)

    accv[0, :] = sx
    accv[1, :] = sy
    pltpu.sync_copy(accv, out_hbm.at[wid])


_SC_CACHE = []


def _sc_sl1(reg, mp, msk):
    if not _SC_CACHE:
        _SC_CACHE.append(functools.partial(
            pl.kernel,
            out_type=jax.ShapeDtypeStruct((_SC_NC * _SC_NS, 2, 16),
                                          jnp.float32),
            mesh=plsc.VectorSubcoreMesh(core_axis_name="c",
                                        subcore_axis_name="s",
                                        num_cores=_SC_NC,
                                        num_subcores=_SC_NS),
            scratch_types=[
                pltpu.VMEM((3, _SC_ROWS, _SC_S), jnp.int32),
                pltpu.VMEM((_SC_ROWS, _SC_S), jnp.float32),
                pltpu.VMEM((2, 8, _SC_ROWS, _SC_S), jnp.float32),
                pltpu.VMEM((2, 16), jnp.float32),
                pltpu.SemaphoreType.DMA,
                pltpu.SemaphoreType.DMA,
                pltpu.SemaphoreType.DMA,
            ],
        )(_sc_sl1_kernel))
    return _SC_CACHE[0](reg, mp, msk)


def kernel(p3_cls, p3_reg, p3_mask, p3_map,
           p4_cls, p4_reg, p4_mask, p4_map,
           p5_cls, p5_reg, p5_mask, p5_map):
    sc = _sc_sl1(p3_reg, p3_map, p3_mask)
    tc = _tc_call(p3_cls, p3_mask, p4_cls, p4_mask, p4_reg, p4_map,
                  p5_cls, p5_mask, p5_reg, p5_map)
    sx3 = jnp.sum(sc[:, 0, :])
    sy3 = jnp.sum(sc[:, 1, :])
    np3 = tc[4]
    has3 = np3 > 0.0
    den3 = jnp.maximum(np3 * float(_K), 1.0)
    l_rx = tc[2] + jnp.where(has3, sx3 / den3, 0.0)
    l_ry = tc[3] + jnp.where(has3, sy3 / den3, 0.0)
    return jnp.stack([tc[0], tc[1], l_rx, l_ry])


# speedup vs baseline: 1.3205x; 1.2802x over previous
import jax
import jax.numpy as jnp
from jax.experimental import pallas as pl
from jax.experimental.pallas import tpu as pltpu

_K = 8
_OHEM_RATIO = 3.0
_NEG_FILL = -1e30
_BISECT_ITERS = 40

_LVL = (
    dict(base=0, nsteps=40, nc=5, bh=32, s=160, total=8 * 160 * 160),
    dict(base=40, nsteps=8, nc=1, bh=80, s=80, total=8 * 80 * 80),
    dict(base=48, nsteps=8, nc=1, bh=40, s=40, total=8 * 40 * 40),
)
_NSTEPS = 56


def _softplus(x):
    return jnp.maximum(x, 0.0) + jnp.log1p(jnp.exp(-jnp.abs(x)))


def _level_body(t, lvl, cls_ref, msk_ref, rx_ref, ry_ref, gx_ref, gy_ref,
                out_ref, accm_ref, accx_ref, accy_ref,
                negce_ref, sel_ref):
    base, nsteps, total = lvl["base"], lvl["nsteps"], lvl["total"]
    step = t - base

    @pl.when(step == 0)
    def _init():
        accm_ref[...] = jnp.zeros_like(accm_ref)
        accx_ref[...] = jnp.zeros_like(accx_ref)
        accy_ref[...] = jnp.zeros_like(accy_ref)

    c = cls_ref[0]
    m = msk_ref[0].astype(jnp.float32)
    tr = m[0]
    tcl = m[1]
    tm = m[2]
    ce_tr = _softplus((c[0] - c[1]) * (2.0 * tr - 1.0))
    ce_tcl = _softplus((c[2] - c[3]) * (2.0 * tcl - 1.0))

    pos = tr * tm
    neg = tm - pos
    negce = jnp.where(neg > 0.0, ce_tr, _NEG_FILL)
    negce_ref[step] = negce

    accm_ref[0] += tm + 64.0 * pos
    accm_ref[1] += ce_tr * pos
    accm_ref[2] += ce_tcl * pos
    accm_ref[3] += ce_tcl
    accm_ref[4] += ce_tr * neg

    wm = pos * (tr + tcl) * 0.125
    dx = jnp.abs(gx_ref[0] - rx_ref[0])
    mx = jnp.minimum(dx, 1.0)
    accx_ref[...] += wm * jnp.sum(dx - mx + 0.5 * mx * mx, axis=0)
    dy = jnp.abs(gy_ref[0] - ry_ref[0])
    my = jnp.minimum(dy, 1.0)
    accy_ref[...] += wm * jnp.sum(dy - my + 0.5 * my * my, axis=0)

    @pl.when(step == nsteps - 1)
    def _finalize():
        packed = accm_ref[0]
        pos_pl = jnp.floor(packed * (1.0 / 64.0))
        n_pos = jnp.sum(pos_pl)
        n_neg_all = jnp.sum(packed - 64.0 * pos_pl) - n_pos
        s_ce_pos = jnp.sum(accm_ref[1])
        s_tcl_pos = jnp.sum(accm_ref[2])
        s_tcl_all = jnp.sum(accm_ref[3])
        s_neg_all = jnp.sum(accm_ref[4])
        sx = jnp.sum(accx_ref[...])
        sy = jnp.sum(accy_ref[...])

        has_pos = n_pos > 0.0
        n_neg = jnp.where(has_pos,
                          jnp.minimum(n_neg_all,
                                      jnp.floor(_OHEM_RATIO * n_pos)),
                          100.0)
        eff = jnp.minimum(n_neg, n_neg_all)
        need_select = eff < n_neg_all

        @pl.when(need_select)
        def _bisect():
            v = negce_ref[...]
            maxv = jnp.maximum(jnp.max(v), 0.0)

            def body(_, carry):
                lo, hi = carry
                mid = 0.5 * (lo + hi)
                cnt = jnp.sum((v > mid).astype(jnp.float32))
                take_lo = cnt >= eff
                return (jnp.where(take_lo, mid, lo),
                        jnp.where(take_lo, hi, mid))

            lo, hi = jax.lax.fori_loop(0, _BISECT_ITERS, body, (0.0, maxv))
            cnt_hi = jnp.sum((v > hi).astype(jnp.float32))
            sum_hi = jnp.sum(jnp.where(v > hi, v, 0.0))
            sel_ref[0] = sum_hi + (eff - cnt_hi) * hi

        loss_neg = jnp.where(need_select, sel_ref[0], s_neg_all)
        loss_pos = jnp.where(has_pos, s_ce_pos, 0.0)
        l_tr = (loss_pos + loss_neg) / (n_pos + n_neg)

        tcl_pos = s_tcl_pos / jnp.maximum(n_pos, 1.0)
        tcl_neg = (s_tcl_all - s_tcl_pos) / jnp.maximum(float(total) - n_pos,
                                                        1.0)
        l_tcl = jnp.where(has_pos, tcl_pos + 0.5 * tcl_neg, 0.0)

        denom = jnp.maximum(n_pos * float(_K), 1.0)
        l_rx = jnp.where(has_pos, sx / denom, 0.0)
        l_ry = jnp.where(has_pos, sy / denom, 0.0)

        if base == 0:
            out_ref[0] = l_tr
            out_ref[1] = l_tcl
            out_ref[2] = l_rx
            out_ref[3] = l_ry
        else:
            out_ref[0] += l_tr
            out_ref[1] += l_tcl
            out_ref[2] += l_rx
            out_ref[3] += l_ry


def _fused_kernel(c3, m3, rx3, ry3, gx3, gy3,
                  c4, m4, rx4, ry4, gx4, gy4,
                  c5, m5, rx5, ry5, gx5, gy5,
                  out_ref,
                  accm3, accx3, accy3, negce3,
                  accm4, accx4, accy4, negce4,
                  accm5, accx5, accy5, negce5,
                  sel_ref):
    t = pl.program_id(0)
    ops = ((c3, m3, rx3, ry3, gx3, gy3, out_ref,
            accm3, accx3, accy3, negce3, sel_ref),
           (c4, m4, rx4, ry4, gx4, gy4, out_ref,
            accm4, accx4, accy4, negce4, sel_ref),
           (c5, m5, rx5, ry5, gx5, gy5, out_ref,
            accm5, accx5, accy5, negce5, sel_ref))
    for lvl, refs in zip(_LVL, ops):
        lo, hi = lvl["base"], lvl["base"] + lvl["nsteps"]

        @pl.when((t >= lo) & (t < hi))
        def _run(lvl=lvl, refs=refs):
            _level_body(t, lvl, *refs)


def _level_specs(lvl):
    base, nc, bh, s = lvl["base"], lvl["nc"], lvl["bh"], lvl["s"]
    last_i = lvl["nsteps"] // nc - 1
    last_j = nc - 1

    def im(cblk):
        def f(t):
            sidx = t - base
            i = jnp.clip(sidx // nc, 0, last_i)
            j = jnp.clip(sidx % nc, 0, last_j)
            j = jnp.where((t >= base) & (sidx // nc <= last_i), j, last_j)
            j = jnp.where(t < base, 0, j)
            i = jnp.where(t < base, 0, i)
            return (i, cblk, j, 0)
        return f

    return [
        pl.BlockSpec((1, 4, bh, s), im(0)),
        pl.BlockSpec((1, 3, bh, s), im(0)),
        pl.BlockSpec((1, _K, bh, s), im(0)),
        pl.BlockSpec((1, _K, bh, s), im(1)),
        pl.BlockSpec((1, _K, bh, s), im(0)),
        pl.BlockSpec((1, _K, bh, s), im(1)),
    ]


def _level_scratch(lvl):
    bh, s, nsteps = lvl["bh"], lvl["s"], lvl["nsteps"]
    return [
        pltpu.VMEM((5, bh, s), jnp.float32),
        pltpu.VMEM((bh, s), jnp.float32),
        pltpu.VMEM((bh, s), jnp.float32),
        pltpu.VMEM((nsteps, bh, s), jnp.float32),
    ]


def kernel(p3_cls, p3_reg, p3_mask, p3_map,
           p4_cls, p4_reg, p4_mask, p4_map,
           p5_cls, p5_reg, p5_mask, p5_map):
    specs = (_level_specs(_LVL[0]) + _level_specs(_LVL[1])
             + _level_specs(_LVL[2]))
    scratch = (_level_scratch(_LVL[0]) + _level_scratch(_LVL[1])
               + _level_scratch(_LVL[2]) + [pltpu.SMEM((1,), jnp.float32)])
    return pl.pallas_call(
        _fused_kernel,
        grid=(_NSTEPS,),
        in_specs=specs,
        out_specs=pl.BlockSpec(memory_space=pltpu.SMEM),
        out_shape=jax.ShapeDtypeStruct((4,), jnp.float32),
        scratch_shapes=scratch,
    )(p3_cls, p3_mask, p3_reg, p3_reg, p3_map, p3_map,
      p4_cls, p4_mask, p4_reg, p4_reg, p4_map, p4_map,
      p5_cls, p5_mask, p5_reg, p5_reg, p5_map, p5_map)
